# baseline (speedup 1.0000x reference)
import functools

import jax
import jax.numpy as jnp
from jax import lax
from jax.experimental import pallas as pl
from jax.experimental.pallas import tpu as pltpu
from jax.experimental.pallas import tpu_sc as plsc

_BATCH = 16384
_ROWS = 2 * _BATCH
_DIM = 64
_NC = 2
_NS = 16
_NW = _NC * _NS
_LANES = 16
_NT0 = 7813
_K = 245
_NWAVE = (_K + 3) // 4
_CAP = 2048
_ZROWS = 36864

_mesh = plsc.VectorSubcoreMesh(
    core_axis_name="c", subcore_axis_name="s", num_cores=_NC, num_subcores=_NS
)


@functools.partial(
    pl.kernel,
    out_type=jax.ShapeDtypeStruct((_ZROWS, 128), jnp.float32),
    mesh=_mesh,
    scratch_types=[
        pltpu.VMEM((2, 16, 128), jnp.int32),
        pltpu.VMEM((_CAP + 16,), jnp.int32),
        pltpu.VMEM((_CAP + 16,), jnp.int32),
        pltpu.VMEM((_CAP + 16,), jnp.int32),
        pltpu.VMEM((_CAP + 16,), jnp.int32),
        pltpu.VMEM((256,), jnp.int32),
        pltpu.VMEM((256,), jnp.int32),
        pltpu.VMEM((272,), jnp.int32),
        pltpu.VMEM((_CAP + 16,), jnp.int32),
        pltpu.VMEM((_CAP + 16,), jnp.int32),
        pltpu.VMEM((_CAP + 16,), jnp.int32),
        pltpu.VMEM((12, 64, 128), jnp.float32),
        pltpu.VMEM((32, 128), jnp.float32),
        pltpu.VMEM((2, 1, 16), jnp.int32),
        pltpu.SemaphoreType.DMA,
        pltpu.SemaphoreType.DMA,
        pltpu.SemaphoreType.DMA,
        pltpu.SemaphoreType.DMA,
    ],
    compiler_params=pltpu.CompilerParams(
        needs_layout_passes=False, use_tc_tiling_on_sc=True),
)
def _hash_gather(sidx_hbm, tablet_hbm, zbuf_hbm,
                 sidx_v, cand_r, cand_k, cand_j, candp, used, slotpos,
                 fetchlist, sel_j, sel_r, sel_s, ring, wavebuf, widx,
                 sem_in, sem_ring, sem_sc, sem_sc2):
    wid = lax.axis_index("s") * _NC + lax.axis_index("c")
    iota = lax.iota(jnp.int32, _LANES)
    ones = jnp.ones((_LANES,), jnp.int32)
    zeros = jnp.zeros((_LANES,), jnp.int32)

    pltpu.async_copy(sidx_hbm.at[pl.ds(0, 16)], sidx_v.at[0], sem_sc2)

    def scan_chunk(c, nc):
        par = c & 1

        @pl.when(c + 1 < 16)
        def _prefetch():
            pltpu.async_copy(
                sidx_hbm.at[pl.ds((c + 1) * 16, 16)], sidx_v.at[(c + 1) & 1],
                sem_sc2)

        pltpu.make_async_copy(
            sidx_hbm.at[pl.ds(0, 16)], sidx_v.at[par], sem_sc2).wait()

        def scan_row(r8, nc):
            for l in range(8):
                rv = sidx_v[par, r8, pl.ds(l * 16, 16)]
                jv = iota + (c * 2048 + r8 * 128 + l * 16)
                t0 = rv >> 7
                m = (t0 & 31) == wid
                cnt = plsc.all_reduce_population_count(m)[0]
                ncs = jnp.minimum(nc, _CAP)
                plsc.store_compressed(cand_r.at[pl.ds(ncs, 16)], rv & 127, mask=m)
                plsc.store_compressed(cand_k.at[pl.ds(ncs, 16)], rv >> 12, mask=m)
                plsc.store_compressed(cand_j.at[pl.ds(ncs, 16)], jv, mask=m)
                nc = nc + cnt
            return nc

        return lax.fori_loop(0, 16, scan_row, nc)

    nc = lax.fori_loop(0, 16, scan_chunk, jnp.int32(0))
    nc = jnp.minimum(nc, _CAP)
    nchk = (nc + 15) >> 4

    for i in range(16):
        used[pl.ds(i * 16, 16)] = zeros

    def mark(c, carry):
        base = c * 16
        kv = plsc.load_gather(cand_k, [iota + base])
        m = iota < (nc - base)
        plsc.store_scatter(used, [kv & 255], ones, mask=m)
        return carry

    lax.fori_loop(0, nchk, mark, jnp.int32(0))

    nf = jnp.int32(0)
    for i in range(16):
        uv = used[pl.ds(i * 16, 16)]
        m = uv != 0
        kvals = iota + i * 16
        pcnt = plsc.all_reduce_population_count(m)[0]
        pref = plsc.cumsum(m.astype(jnp.int32))
        plsc.store_compressed(fetchlist.at[pl.ds(nf, 16)], kvals, mask=m)
        plsc.store_scatter(slotpos, [kvals], nf + pref - 1, mask=m)
        nf = nf + pcnt

    def fill_candp(c, carry):
        base = c * 16
        kv = plsc.load_gather(cand_k, [iota + base])
        pv = plsc.load_gather(slotpos, [kv & 255])
        candp[pl.ds(base, 16)] = pv
        return carry

    lax.fori_loop(0, nchk, fill_candp, jnp.int32(0))

    def fire(w, par, sem):
        for b in range(4):
            fi = w * 4 + b

            @pl.when(fi < nf)
            def _fire():
                kv16 = fetchlist[pl.ds(fi, 16)]
                col = (kv16[0] * 32 + wid) * 128
                col = pl.multiple_of(col, 128)
                pltpu.async_copy(
                    tablet_hbm.at[:, pl.ds(col, 128)], ring.at[par * 4 + b],
                    sem)

    def process(w, par, sem, gct):
        for b in range(4):

            @pl.when(w * 4 + b < nf)
            def _drain():
                pltpu.make_async_copy(
                    tablet_hbm.at[:, pl.ds(0, 128)], ring.at[par * 4 + b],
                    sem).wait()

        def select(c, p2):
            base = c * 16
            pv = plsc.load_gather(candp, [iota + base])
            m = (iota < (nc - base)) & ((pv >> 2) == w)
            cnt = plsc.all_reduce_population_count(m)[0]

            @pl.when(cnt > 0)
            def _emit():
                p2s = jnp.minimum(p2, _CAP)
                plsc.store_compressed(
                    sel_j.at[pl.ds(p2s, 16)],
                    plsc.load_gather(cand_j, [iota + base]), mask=m)
                plsc.store_compressed(
                    sel_r.at[pl.ds(p2s, 16)],
                    plsc.load_gather(cand_r, [iota + base]), mask=m)
                plsc.store_compressed(
                    sel_s.at[pl.ds(p2s, 16)], pv & 3, mask=m)

            return p2 + cnt

        ptr2 = lax.fori_loop(0, nchk, select, jnp.int32(0))
        ptr2 = jnp.minimum(ptr2, _CAP)

        def group(g, gct):
            base = g * 16
            lane_m = iota < (ptr2 - base)
            jv = plsc.load_gather(sel_j, [iota + base])
            rv = plsc.load_gather(sel_r, [iota + base])
            sv = plsc.load_gather(sel_s, [iota + base]) + par * 4
            gpar = gct & 1

            @pl.when(gct >= 2)
            def _wait_prev():
                pltpu.make_async_copy(
                    wavebuf.at[pl.ds(0, 16)], zbuf_hbm.at[widx.at[0, 0]],
                    sem_sc).wait()

            rowv = gpar * 16 + iota
            for d in range(_DIM):
                dspl = jnp.full((_LANES,), d, jnp.int32)
                vals = plsc.load_gather(
                    ring, [sv, dspl, rv], mask=lane_m)
                plsc.store_scatter(wavebuf, [rowv, dspl], vals)
            idxv = jnp.where(lane_m, jv, _ROWS + wid)
            widx[gpar, 0, :] = idxv
            pltpu.async_copy(
                wavebuf.at[pl.ds(gpar * 16, 16)],
                zbuf_hbm.at[widx.at[gpar, 0]], sem_sc)
            return gct + 1

        ng = (ptr2 + 15) >> 4
        return lax.fori_loop(0, ng, group, gct)

    fire(jnp.int32(0), 0, sem_ring)
    fire(jnp.int32(1), 1, sem_in)

    def wave_triplet(t, gct):
        w0 = t * 3
        fire(w0 + 2, 2, sem_sc2)
        gct = process(w0, 0, sem_ring, gct)
        fire(w0 + 3, 0, sem_ring)
        gct = process(w0 + 1, 1, sem_in, gct)
        fire(w0 + 4, 1, sem_in)
        gct = process(w0 + 2, 2, sem_sc2, gct)
        return gct

    gct = lax.fori_loop(0, (_NWAVE + 2) // 3, wave_triplet, jnp.int32(0))

    def drain_sc(i, carry):
        @pl.when(i < jnp.minimum(gct, 2))
        def _d():
            pltpu.make_async_copy(
                wavebuf.at[pl.ds(0, 16)], zbuf_hbm.at[widx.at[0, 0]],
                sem_sc).wait()
        return carry

    lax.fori_loop(0, 2, drain_sc, jnp.int32(0))


def _tc_body(s_ref, z_ref, o_ref):
    x = z_ref[...]
    y = x.reshape(x.shape[0] // 2, 2, 128)
    d = y[:, 0, :] - y[:, 1, :]
    dmask = lax.broadcasted_iota(jnp.int32, d.shape, 1) < _DIM
    d2 = jnp.sum(jnp.where(dmask, d * d, 0.0), axis=-1) + 1e-12
    dist = jnp.sqrt(d2)
    sim = jnp.exp(-s_ref[4] * dist)
    t = jnp.exp(-s_ref[3] * (sim - s_ref[2]))
    o_ref[...] = s_ref[0] + (s_ref[1] - s_ref[0]) / (1.0 + t)


_BROWS = 4096


def kernel(stimulus_set, percept_table, lower, upper, midpoint, rate, beta):
    sidx = stimulus_set.astype(jnp.int32).reshape(256, 128)
    tablet = percept_table.T
    zbuf = _hash_gather(sidx, tablet)
    zero = jnp.float32(0)
    params = jnp.stack([
        jnp.float32(lower), jnp.float32(upper), jnp.float32(midpoint),
        jnp.float32(rate), jnp.float32(beta), zero, zero, zero,
    ])
    prob = pl.pallas_call(
        _tc_body,
        grid=(_ROWS // _BROWS,),
        in_specs=[
            pl.BlockSpec(memory_space=pltpu.SMEM),
            pl.BlockSpec((_BROWS, 128), lambda i: (i, 0)),
        ],
        out_specs=pl.BlockSpec((_BROWS // 2,), lambda i: (i,)),
        out_shape=jax.ShapeDtypeStruct((_BATCH,), jnp.float32),
    )(params, zbuf)
    return prob.reshape(_BATCH, 1)


# speedup vs baseline: 1.0142x; 1.0142x over previous
import functools

import jax
import jax.numpy as jnp
from jax import lax
from jax.experimental import pallas as pl
from jax.experimental.pallas import tpu as pltpu
from jax.experimental.pallas import tpu_sc as plsc

_BATCH = 16384
_ROWS = 2 * _BATCH
_DIM = 64
_NC = 2
_NS = 16
_NW = _NC * _NS
_LANES = 16
_NT0 = 7813
_K = 245
_NWAVE = (_K + 3) // 4
_CAP = 4096
_ZROWS = 36864

_mesh = plsc.VectorSubcoreMesh(
    core_axis_name="c", subcore_axis_name="s", num_cores=_NC, num_subcores=_NS
)


@functools.partial(
    pl.kernel,
    out_type=jax.ShapeDtypeStruct((_ZROWS, 128), jnp.float32),
    mesh=_mesh,
    scratch_types=[
        pltpu.VMEM((2, 16, 128), jnp.int32),
        pltpu.VMEM((_CAP + 16,), jnp.int32),
        pltpu.VMEM((_CAP + 16,), jnp.int32),
        pltpu.VMEM((_CAP + 16,), jnp.int32),
        pltpu.VMEM((_CAP + 16,), jnp.int32),
        pltpu.VMEM((256,), jnp.int32),
        pltpu.VMEM((256,), jnp.int32),
        pltpu.VMEM((272,), jnp.int32),
        pltpu.VMEM((_CAP + 16,), jnp.int32),
        pltpu.VMEM((_CAP + 16,), jnp.int32),
        pltpu.VMEM((_CAP + 16,), jnp.int32),
        pltpu.VMEM((8, 64, 128), jnp.float32),
        pltpu.VMEM((32, 128), jnp.float32),
        pltpu.VMEM((2, 1, 16), jnp.int32),
        pltpu.SemaphoreType.DMA,
        pltpu.SemaphoreType.DMA,
        pltpu.SemaphoreType.DMA,
        pltpu.SemaphoreType.DMA,
    ],
    compiler_params=pltpu.CompilerParams(
        needs_layout_passes=False, use_tc_tiling_on_sc=True),
)
def _hash_gather(sidx_hbm, tablet_hbm, zbuf_hbm,
                 sidx_v, cand_r, cand_k, cand_j, candp, used, slotpos,
                 fetchlist, sel_j, sel_r, sel_s, ring, wavebuf, widx,
                 sem_in, sem_ring, sem_sc, sem_sc2):
    wid = lax.axis_index("s") * _NC + lax.axis_index("c")
    iota = lax.iota(jnp.int32, _LANES)
    ones = jnp.ones((_LANES,), jnp.int32)
    zeros = jnp.zeros((_LANES,), jnp.int32)

    pltpu.async_copy(sidx_hbm.at[pl.ds(0, 16)], sidx_v.at[0], sem_sc2)

    def scan_chunk(c, nc):
        par = c & 1

        @pl.when(c + 1 < 16)
        def _prefetch():
            pltpu.async_copy(
                sidx_hbm.at[pl.ds((c + 1) * 16, 16)], sidx_v.at[(c + 1) & 1],
                sem_sc2)

        pltpu.make_async_copy(
            sidx_hbm.at[pl.ds(0, 16)], sidx_v.at[par], sem_sc2).wait()

        def scan_row(r8, nc):
            for l in range(8):
                rv = sidx_v[par, r8, pl.ds(l * 16, 16)]
                jv = iota + (c * 2048 + r8 * 128 + l * 16)
                t0 = rv >> 7
                m = (t0 & 31) == wid
                cnt = plsc.all_reduce_population_count(m)[0]
                ncs = jnp.minimum(nc, _CAP)
                plsc.store_compressed(cand_r.at[pl.ds(ncs, 16)], rv & 127, mask=m)
                plsc.store_compressed(cand_k.at[pl.ds(ncs, 16)], rv >> 12, mask=m)
                plsc.store_compressed(cand_j.at[pl.ds(ncs, 16)], jv, mask=m)
                nc = nc + cnt
            return nc

        return lax.fori_loop(0, 16, scan_row, nc)

    nc = lax.fori_loop(0, 16, scan_chunk, jnp.int32(0))
    nc = jnp.minimum(nc, _CAP)
    nchk = (nc + 15) >> 4

    for i in range(16):
        used[pl.ds(i * 16, 16)] = zeros

    def mark(c, carry):
        base = c * 16
        kv = plsc.load_gather(cand_k, [iota + base])
        m = iota < (nc - base)
        plsc.store_scatter(used, [kv & 255], ones, mask=m)
        return carry

    lax.fori_loop(0, nchk, mark, jnp.int32(0))

    nf = jnp.int32(0)
    for i in range(16):
        uv = used[pl.ds(i * 16, 16)]
        m = uv != 0
        kvals = iota + i * 16
        pcnt = plsc.all_reduce_population_count(m)[0]
        pref = plsc.cumsum(m.astype(jnp.int32))
        plsc.store_compressed(fetchlist.at[pl.ds(nf, 16)], kvals, mask=m)
        plsc.store_scatter(slotpos, [kvals], nf + pref - 1, mask=m)
        nf = nf + pcnt

    def fill_candp(c, carry):
        base = c * 16
        kv = plsc.load_gather(cand_k, [iota + base])
        pv = plsc.load_gather(slotpos, [kv & 255])
        candp[pl.ds(base, 16)] = pv
        return carry

    lax.fori_loop(0, nchk, fill_candp, jnp.int32(0))

    def fire(w, par, sem):
        for b in range(4):
            fi = w * 4 + b

            @pl.when(fi < nf)
            def _fire():
                kv16 = fetchlist[pl.ds(fi, 16)]
                col = (kv16[0] * 32 + wid) * 128
                col = pl.multiple_of(col, 128)
                pltpu.async_copy(
                    tablet_hbm.at[:, pl.ds(col, 128)], ring.at[par * 4 + b],
                    sem)

    def process(w, par, sem, gct):
        for b in range(4):

            @pl.when(w * 4 + b < nf)
            def _drain():
                pltpu.make_async_copy(
                    tablet_hbm.at[:, pl.ds(0, 128)], ring.at[par * 4 + b],
                    sem).wait()

        def select(c, p2):
            base = c * 16
            pv = plsc.load_gather(candp, [iota + base])
            m = (iota < (nc - base)) & ((pv >> 2) == w)
            cnt = plsc.all_reduce_population_count(m)[0]

            @pl.when(cnt > 0)
            def _emit():
                p2s = jnp.minimum(p2, _CAP)
                plsc.store_compressed(
                    sel_j.at[pl.ds(p2s, 16)],
                    plsc.load_gather(cand_j, [iota + base]), mask=m)
                plsc.store_compressed(
                    sel_r.at[pl.ds(p2s, 16)],
                    plsc.load_gather(cand_r, [iota + base]), mask=m)
                plsc.store_compressed(
                    sel_s.at[pl.ds(p2s, 16)], pv & 3, mask=m)

            return p2 + cnt

        ptr2 = lax.fori_loop(0, nchk, select, jnp.int32(0))
        ptr2 = jnp.minimum(ptr2, _CAP)

        def group(g, gct):
            base = g * 16
            lane_m = iota < (ptr2 - base)
            jv = plsc.load_gather(sel_j, [iota + base])
            rv = plsc.load_gather(sel_r, [iota + base])
            sv = plsc.load_gather(sel_s, [iota + base]) + par * 4
            gpar = gct & 1

            @pl.when(gct >= 2)
            def _wait_prev():
                pltpu.make_async_copy(
                    wavebuf.at[pl.ds(0, 16)], zbuf_hbm.at[widx.at[0, 0]],
                    sem_sc).wait()

            rowv = gpar * 16 + iota
            for d in range(_DIM):
                dspl = jnp.full((_LANES,), d, jnp.int32)
                vals = plsc.load_gather(
                    ring, [sv, dspl, rv], mask=lane_m)
                plsc.store_scatter(wavebuf, [rowv, dspl], vals)
            idxv = jnp.where(lane_m, jv, _ROWS + wid)
            widx[gpar, 0, :] = idxv
            pltpu.async_copy(
                wavebuf.at[pl.ds(gpar * 16, 16)],
                zbuf_hbm.at[widx.at[gpar, 0]], sem_sc)
            return gct + 1

        ng = (ptr2 + 15) >> 4
        return lax.fori_loop(0, ng, group, gct)

    fire(jnp.int32(0), 0, sem_ring)

    def wave_pair(t, gct):
        w0 = t * 2
        fire(w0 + 1, 1, sem_in)
        gct = process(w0, 0, sem_ring, gct)
        fire(w0 + 2, 0, sem_ring)
        gct = process(w0 + 1, 1, sem_in, gct)
        return gct

    gct = lax.fori_loop(0, _NWAVE // 2, wave_pair, jnp.int32(0))

    def drain_sc(i, carry):
        @pl.when(i < jnp.minimum(gct, 2))
        def _d():
            pltpu.make_async_copy(
                wavebuf.at[pl.ds(0, 16)], zbuf_hbm.at[widx.at[0, 0]],
                sem_sc).wait()
        return carry

    lax.fori_loop(0, 2, drain_sc, jnp.int32(0))


def _tc_body(s_ref, z_ref, o_ref):
    x = z_ref[...]
    y = x.reshape(x.shape[0] // 2, 2, 128)
    d = y[:, 0, :] - y[:, 1, :]
    dmask = lax.broadcasted_iota(jnp.int32, d.shape, 1) < _DIM
    d2 = jnp.sum(jnp.where(dmask, d * d, 0.0), axis=-1) + 1e-12
    dist = jnp.sqrt(d2)
    sim = jnp.exp(-s_ref[4] * dist)
    t = jnp.exp(-s_ref[3] * (sim - s_ref[2]))
    o_ref[...] = s_ref[0] + (s_ref[1] - s_ref[0]) / (1.0 + t)


_BROWS = 4096


def kernel(stimulus_set, percept_table, lower, upper, midpoint, rate, beta):
    sidx = stimulus_set.astype(jnp.int32).reshape(256, 128)
    tablet = percept_table.T
    zbuf = _hash_gather(sidx, tablet)
    zero = jnp.float32(0)
    params = jnp.stack([
        jnp.float32(lower), jnp.float32(upper), jnp.float32(midpoint),
        jnp.float32(rate), jnp.float32(beta), zero, zero, zero,
    ])
    prob = pl.pallas_call(
        _tc_body,
        grid=(_ROWS // _BROWS,),
        in_specs=[
            pl.BlockSpec(memory_space=pltpu.SMEM),
            pl.BlockSpec((_BROWS, 128), lambda i: (i, 0)),
        ],
        out_specs=pl.BlockSpec((_BROWS // 2,), lambda i: (i,)),
        out_shape=jax.ShapeDtypeStruct((_BATCH,), jnp.float32),
    )(params, zbuf)
    return prob.reshape(_BATCH, 1)
